# Initial kernel scaffold; baseline (speedup 1.0000x reference)
import jax
import jax.numpy as jnp
from jax.experimental import pallas as pl


def kernel(context_features, params_inter, params_spk, params_oth, fc_w, fc_b, context_lengths, context_speaker_ids, roles):
    raise NotImplementedError("write your pallas kernel here")


# speedup vs baseline: 7.8174x; 7.8174x over previous
import jax
import jax.numpy as jnp
from jax.experimental import pallas as pl
from jax.experimental.pallas import tpu as pltpu

CT = 32


def _fused_body(Bb, Hh, nc,
                x_ref, code_ref,
                wi1, wh1, bi1, bh1, wi2, wh2, bi2, bh2,
                wis1, whs1, bis1, bhs1, wis2, whs2, bis2, bhs2,
                wio1, who1, bio1, bho1, wio2, who2, bio2, bho2,
                fcw, fcb,
                out_ref,
                gA, gB, y1, y2, ys1, yo1,
                h1, h2, hs1, hs2, ho1, ho2, any_s, any_o):
    c = pl.program_id(0)
    f32 = jnp.float32

    @pl.when(c == 0)
    def _init():
        for r in (h1, h2, hs1, hs2, ho1, ho2, any_s, any_o):
            r[...] = jnp.zeros_like(r)

    def dense(src_ref, w_ref, b_ref, dst_ref):
        Xm = src_ref[...].reshape(CT * Bb, -1)
        dst_ref[...] = (
            jnp.dot(Xm, w_ref[...], preferred_element_type=f32) + b_ref[0:1, :]
        ).reshape(CT, Bb, 3 * Hh)

    def cell(gi, gh, h):
        r = jax.nn.sigmoid(gi[:, :Hh] + gh[:, :Hh])
        z = jax.nn.sigmoid(gi[:, Hh:2 * Hh] + gh[:, Hh:2 * Hh])
        n = jnp.tanh(gi[:, 2 * Hh:] + r * gh[:, 2 * Hh:])
        return (1.0 - z) * n + z * h

    def scan_valid(gi_ref, y_ref, h_ref, w_ref, b_ref):
        def step(t, carry):
            h = h_ref[...]
            gh = jnp.dot(h, w_ref[...], preferred_element_type=f32) + b_ref[0:1, :]
            hn = cell(gi_ref[t], gh, h)
            hv = jnp.where(code_ref[t] != 0.0, hn, h)
            h_ref[...] = hv
            y_ref[t] = hv
            return carry
        jax.lax.fori_loop(0, CT, step, 0)

    def scan_pair(gis_ref, gio_ref, hsr, hor, ws, bs, wo, bo, ysr=None, yor=None):
        def step(t, carry):
            code = code_ref[t]
            hs = hsr[...]
            ho = hor[...]
            ghs = jnp.dot(hs, ws[...], preferred_element_type=f32) + bs[0:1, :]
            gho = jnp.dot(ho, wo[...], preferred_element_type=f32) + bo[0:1, :]
            hsn = cell(gis_ref[t], ghs, hs)
            hon = cell(gio_ref[t], gho, ho)
            hsv = jnp.where(code > 0.0, hsn, hs)
            hov = jnp.where(code < 0.0, hon, ho)
            hsr[...] = hsv
            hor[...] = hov
            if ysr is not None:
                ysr[t] = hsv
                yor[t] = hov
            return carry
        jax.lax.fori_loop(0, CT, step, 0)

    dense(x_ref, wi1, bi1, gA)
    scan_valid(gA, y1, h1, wh1, bh1)
    dense(y1, wi2, bi2, gA)
    scan_valid(gA, y2, h2, wh2, bh2)
    dense(y2, wis1, bis1, gA)
    dense(y2, wio1, bio1, gB)
    scan_pair(gA, gB, hs1, ho1, whs1, bhs1, who1, bho1, ys1, yo1)
    dense(ys1, wis2, bis2, gA)
    dense(yo1, wio2, bio2, gB)
    scan_pair(gA, gB, hs2, ho2, whs2, bhs2, who2, bho2)

    codes = code_ref[...]
    any_s[...] = jnp.maximum(any_s[...], jnp.max((codes > 0.0).astype(f32), axis=0))
    any_o[...] = jnp.maximum(any_o[...], jnp.max((codes < 0.0).astype(f32), axis=0))

    @pl.when(c == nc - 1)
    def _final():
        zero1 = jnp.zeros((1, Hh), f32)

        def fall2(bi_1, bh_1, wi_2, bi_2, bh_2):
            f1 = cell(bi_1[0:1, :], bh_1[0:1, :], zero1)
            gi = jnp.dot(f1, wi_2[...], preferred_element_type=f32) + bi_2[0:1, :]
            return cell(gi, bh_2[0:1, :], zero1)

        fs = fall2(bis1, bhs1, wis2, bis2, bhs2)
        fo = fall2(bio1, bho1, wio2, bio2, bho2)
        hS = jnp.where(any_s[...] > 0.0, hs2[...], fs)
        hO = jnp.where(any_o[...] > 0.0, ho2[...], fo)
        hcat = jnp.concatenate([hS, hO, h2[...]], axis=1)
        out_ref[...] = jnp.dot(hcat, fcw[...], preferred_element_type=f32) + fcb[...]


def kernel(context_features, params_inter, params_spk, params_oth, fc_w, fc_b,
           context_lengths, context_speaker_ids, roles):
    f32 = jnp.float32
    Bb, T, D = context_features.shape
    Hh = params_inter[0][1].shape[1]
    C = fc_w.shape[0]
    nc = T // CT

    x = jnp.transpose(context_features, (1, 0, 2)).astype(f32)

    lengths = jnp.asarray(context_lengths)
    sid = jnp.asarray(context_speaker_ids)
    roles_a = jnp.asarray(roles)
    t_idx = jnp.arange(T)
    valid = t_idx[:, None] < lengths[None, :]
    match = sid.T == roles_a[None, :]
    code = jnp.where(valid, jnp.where(match, 1.0, -1.0), 0.0).astype(f32)
    code_b = jnp.broadcast_to(code[:, :, None], (T, Bb, Hh))

    def prep(p):
        W_ih, W_hh, b_ih, b_hh = p
        return (W_ih.T.astype(f32), W_hh.T.astype(f32),
                jnp.broadcast_to(b_ih[None, :].astype(f32), (Bb, 3 * Hh)),
                jnp.broadcast_to(b_hh[None, :].astype(f32), (Bb, 3 * Hh)))

    layers = [prep(p) for p in (params_inter + params_spk + params_oth)]
    w_args = [a for lay in layers for a in lay]

    fcw_pad = jnp.zeros((3 * Hh, 128), f32).at[:, :C].set(fc_w.T.astype(f32))
    fcb_pad = jnp.broadcast_to(
        jnp.zeros((128,), f32).at[:C].set(fc_b.astype(f32))[None, :], (Bb, 128))

    full2d = lambda a: pl.BlockSpec(a.shape, lambda c: (0, 0))
    in_specs = [
        pl.BlockSpec((CT, Bb, D), lambda c: (c, 0, 0)),
        pl.BlockSpec((CT, Bb, Hh), lambda c: (c, 0, 0)),
    ] + [full2d(a) for a in w_args] + [full2d(fcw_pad), full2d(fcb_pad)]

    scratch = (
        [pltpu.VMEM((CT, Bb, 3 * Hh), f32)] * 2
        + [pltpu.VMEM((CT, Bb, Hh), f32)] * 4
        + [pltpu.VMEM((Bb, Hh), f32)] * 8
    )

    import functools
    body = functools.partial(_fused_body, Bb, Hh, nc)

    out = pl.pallas_call(
        body,
        grid=(nc,),
        in_specs=in_specs,
        out_specs=pl.BlockSpec((Bb, 128), lambda c: (0, 0)),
        out_shape=jax.ShapeDtypeStruct((Bb, 128), f32),
        scratch_shapes=scratch,
        compiler_params=pltpu.CompilerParams(
            dimension_semantics=("arbitrary",),
            vmem_limit_bytes=100 * 1024 * 1024,
        ),
    )(x, code_b, *w_args, fcw_pad, fcb_pad)

    return out[:, :C]


# speedup vs baseline: 7.9766x; 1.0204x over previous
import jax
import jax.numpy as jnp
from jax.experimental import pallas as pl
from jax.experimental.pallas import tpu as pltpu

CT = 32


def _fused_body(Bb, Hh, nc,
                x_ref, code_ref,
                wi1, wh1, bi1, bh1, wi2, wh2, bi2, bh2,
                wis1, whs1, bis1, bhs1, wis2, whs2, bis2, bhs2,
                wio1, who1, bio1, bho1, wio2, who2, bio2, bho2,
                fcw, fcb,
                out_ref,
                gA, gB, y1, y2, ys1, yo1,
                h1, h2, hs1, hs2, ho1, ho2, any_s, any_o):
    c = pl.program_id(0)
    f32 = jnp.float32

    @pl.when(c == 0)
    def _init():
        for r in (h1, h2, hs1, hs2, ho1, ho2, any_s, any_o):
            r[...] = jnp.zeros_like(r)

    bf16 = jnp.bfloat16

    def dense(src_ref, w_ref, b_ref, dst_ref):
        Xm = src_ref[...].reshape(CT * Bb, -1).astype(bf16)
        dst_ref[...] = (
            jnp.dot(Xm, w_ref[...], preferred_element_type=f32) + b_ref[0:1, :]
        ).reshape(CT, Bb, 3 * Hh)

    def cell(gi, gh, h):
        r = jax.nn.sigmoid(gi[:, :Hh] + gh[:, :Hh])
        z = jax.nn.sigmoid(gi[:, Hh:2 * Hh] + gh[:, Hh:2 * Hh])
        n = jnp.tanh(gi[:, 2 * Hh:] + r * gh[:, 2 * Hh:])
        return (1.0 - z) * n + z * h

    def scan_valid(gi_ref, y_ref, h_ref, w_ref, b_ref):
        def step(t, carry):
            h = h_ref[...]
            gh = jnp.dot(h.astype(bf16), w_ref[...], preferred_element_type=f32) + b_ref[0:1, :]
            hn = cell(gi_ref[t], gh, h)
            hv = jnp.where(code_ref[t] != 0.0, hn, h)
            h_ref[...] = hv
            y_ref[t] = hv
            return carry
        jax.lax.fori_loop(0, CT, step, 0)

    def scan_pair(gis_ref, gio_ref, hsr, hor, ws, bs, wo, bo, ysr=None, yor=None):
        def step(t, carry):
            code = code_ref[t]
            hs = hsr[...]
            ho = hor[...]
            ghs = jnp.dot(hs.astype(bf16), ws[...], preferred_element_type=f32) + bs[0:1, :]
            gho = jnp.dot(ho.astype(bf16), wo[...], preferred_element_type=f32) + bo[0:1, :]
            hsn = cell(gis_ref[t], ghs, hs)
            hon = cell(gio_ref[t], gho, ho)
            hsv = jnp.where(code > 0.0, hsn, hs)
            hov = jnp.where(code < 0.0, hon, ho)
            hsr[...] = hsv
            hor[...] = hov
            if ysr is not None:
                ysr[t] = hsv
                yor[t] = hov
            return carry
        jax.lax.fori_loop(0, CT, step, 0)

    dense(x_ref, wi1, bi1, gA)
    scan_valid(gA, y1, h1, wh1, bh1)
    dense(y1, wi2, bi2, gA)
    scan_valid(gA, y2, h2, wh2, bh2)
    dense(y2, wis1, bis1, gA)
    dense(y2, wio1, bio1, gB)
    scan_pair(gA, gB, hs1, ho1, whs1, bhs1, who1, bho1, ys1, yo1)
    dense(ys1, wis2, bis2, gA)
    dense(yo1, wio2, bio2, gB)
    scan_pair(gA, gB, hs2, ho2, whs2, bhs2, who2, bho2)

    codes = code_ref[...]
    any_s[...] = jnp.maximum(any_s[...], jnp.max((codes > 0.0).astype(f32), axis=0))
    any_o[...] = jnp.maximum(any_o[...], jnp.max((codes < 0.0).astype(f32), axis=0))

    @pl.when(c == nc - 1)
    def _final():
        zero1 = jnp.zeros((1, Hh), f32)

        def fall2(bi_1, bh_1, wi_2, bi_2, bh_2):
            f1 = cell(bi_1[0:1, :], bh_1[0:1, :], zero1)
            gi = jnp.dot(f1.astype(bf16), wi_2[...], preferred_element_type=f32) + bi_2[0:1, :]
            return cell(gi, bh_2[0:1, :], zero1)

        fs = fall2(bis1, bhs1, wis2, bis2, bhs2)
        fo = fall2(bio1, bho1, wio2, bio2, bho2)
        hS = jnp.where(any_s[...] > 0.0, hs2[...], fs)
        hO = jnp.where(any_o[...] > 0.0, ho2[...], fo)
        hcat = jnp.concatenate([hS, hO, h2[...]], axis=1)
        out_ref[...] = jnp.dot(hcat, fcw[...], preferred_element_type=f32) + fcb[...]


def kernel(context_features, params_inter, params_spk, params_oth, fc_w, fc_b,
           context_lengths, context_speaker_ids, roles):
    f32 = jnp.float32
    Bb, T, D = context_features.shape
    Hh = params_inter[0][1].shape[1]
    C = fc_w.shape[0]
    nc = T // CT

    x = jnp.transpose(context_features, (1, 0, 2)).astype(f32)

    lengths = jnp.asarray(context_lengths)
    sid = jnp.asarray(context_speaker_ids)
    roles_a = jnp.asarray(roles)
    t_idx = jnp.arange(T)
    valid = t_idx[:, None] < lengths[None, :]
    match = sid.T == roles_a[None, :]
    code = jnp.where(valid, jnp.where(match, 1.0, -1.0), 0.0).astype(f32)
    code_b = jnp.broadcast_to(code[:, :, None], (T, Bb, Hh))

    def prep(p):
        W_ih, W_hh, b_ih, b_hh = p
        return (W_ih.T.astype(jnp.bfloat16), W_hh.T.astype(jnp.bfloat16),
                jnp.broadcast_to(b_ih[None, :].astype(f32), (Bb, 3 * Hh)),
                jnp.broadcast_to(b_hh[None, :].astype(f32), (Bb, 3 * Hh)))

    layers = [prep(p) for p in (params_inter + params_spk + params_oth)]
    w_args = [a for lay in layers for a in lay]

    fcw_pad = jnp.zeros((3 * Hh, 128), f32).at[:, :C].set(fc_w.T.astype(f32))
    fcb_pad = jnp.broadcast_to(
        jnp.zeros((128,), f32).at[:C].set(fc_b.astype(f32))[None, :], (Bb, 128))

    full2d = lambda a: pl.BlockSpec(a.shape, lambda c: (0, 0))
    in_specs = [
        pl.BlockSpec((CT, Bb, D), lambda c: (c, 0, 0)),
        pl.BlockSpec((CT, Bb, Hh), lambda c: (c, 0, 0)),
    ] + [full2d(a) for a in w_args] + [full2d(fcw_pad), full2d(fcb_pad)]

    scratch = (
        [pltpu.VMEM((CT, Bb, 3 * Hh), f32)] * 2
        + [pltpu.VMEM((CT, Bb, Hh), f32)] * 4
        + [pltpu.VMEM((Bb, Hh), f32)] * 8
    )

    import functools
    body = functools.partial(_fused_body, Bb, Hh, nc)

    out = pl.pallas_call(
        body,
        grid=(nc,),
        in_specs=in_specs,
        out_specs=pl.BlockSpec((Bb, 128), lambda c: (0, 0)),
        out_shape=jax.ShapeDtypeStruct((Bb, 128), f32),
        scratch_shapes=scratch,
        compiler_params=pltpu.CompilerParams(
            dimension_semantics=("arbitrary",),
            vmem_limit_bytes=100 * 1024 * 1024,
        ),
    )(x, code_b, *w_args, fcw_pad, fcb_pad)

    return out[:, :C]


# speedup vs baseline: 8.5858x; 1.0764x over previous
import functools

import jax
import jax.numpy as jnp
from jax.experimental import pallas as pl
from jax.experimental.pallas import tpu as pltpu

CT = 32


def _fused_body(Bb, Hh, nc,
                x_ref, code0_ref, code1_ref, code2_ref, code3_ref,
                wi1, wh1, bi1, bh1, wi2, wh2, bi2, bh2,
                wis1, whs1, bis1, bhs1, wis2, whs2, bis2, bhs2,
                wio1, who1, bio1, bho1, wio2, who2, bio2, bho2,
                fcw, fcb,
                out_ref,
                gA, gB, gC, gD, gE, gF,
                y1, y2, ys1, yo1,
                h1, h2, hs1, hs2, ho1, ho2, any_s, any_o):
    c = pl.program_id(0)
    f32 = jnp.float32
    bf16 = jnp.bfloat16
    p = jax.lax.rem(c, 2)
    q = 1 - p

    @pl.when(c == 0)
    def _init():
        for r in (h1, h2, hs1, hs2, ho1, ho2, any_s, any_o, y1, y2, ys1, yo1):
            r[...] = jnp.zeros_like(r)

    def dense(src, w_ref, b_ref, dst_ref):
        Xm = src.reshape(CT * Bb, -1).astype(bf16)
        dst_ref[...] = (
            jnp.dot(Xm, w_ref[...], preferred_element_type=f32) + b_ref[0:1, :]
        ).reshape(CT, Bb, 3 * Hh)

    dense(x_ref[...], wi1, bi1, gA)
    dense(y1[q], wi2, bi2, gB)
    dense(y2[q], wis1, bis1, gC)
    dense(y2[q], wio1, bio1, gD)
    dense(ys1[q], wis2, bis2, gE)
    dense(yo1[q], wio2, bio2, gF)

    a0 = c < nc
    a1 = (c >= 1) & (c < nc + 1)
    a2 = (c >= 2) & (c < nc + 2)
    a3 = (c >= 3) & (c < nc + 3)

    def cell(gi, gh, h):
        r = jax.nn.sigmoid(gi[:, :Hh] + gh[:, :Hh])
        z = jax.nn.sigmoid(gi[:, Hh:2 * Hh] + gh[:, Hh:2 * Hh])
        n = jnp.tanh(gi[:, 2 * Hh:] + r * gh[:, 2 * Hh:])
        return (1.0 - z) * n + z * h

    def chain(gi_ref, t, h_ref, w_ref, b_ref, m):
        h = h_ref[...]
        gh = jnp.dot(h.astype(bf16), w_ref[...], preferred_element_type=f32) + b_ref[0:1, :]
        hv = jnp.where(m, cell(gi_ref[t], gh, h), h)
        h_ref[...] = hv
        return hv

    def step(t, carry):
        c0 = code0_ref[t]
        c1 = code1_ref[t]
        c2 = code2_ref[t]
        c3 = code3_ref[t]
        y1[p, t] = chain(gA, t, h1, wh1, bh1, (c0 != 0.0) & a0)
        y2[p, t] = chain(gB, t, h2, wh2, bh2, (c1 != 0.0) & a1)
        ys1[p, t] = chain(gC, t, hs1, whs1, bhs1, (c2 > 0.0) & a2)
        yo1[p, t] = chain(gD, t, ho1, who1, bho1, (c2 < 0.0) & a2)
        chain(gE, t, hs2, whs2, bhs2, (c3 > 0.0) & a3)
        chain(gF, t, ho2, who2, bho2, (c3 < 0.0) & a3)
        return carry

    jax.lax.fori_loop(0, CT, step, 0)

    codes = code0_ref[...]
    any_s[...] = jnp.maximum(any_s[...], jnp.max((codes > 0.0).astype(f32), axis=0))
    any_o[...] = jnp.maximum(any_o[...], jnp.max((codes < 0.0).astype(f32), axis=0))

    @pl.when(c == nc + 2)
    def _final():
        zero1 = jnp.zeros((1, Hh), f32)

        def fall2(bi_1, bh_1, wi_2, bi_2, bh_2):
            f1 = cell(bi_1[0:1, :], bh_1[0:1, :], zero1)
            gi = jnp.dot(f1.astype(bf16), wi_2[...], preferred_element_type=f32) + bi_2[0:1, :]
            return cell(gi, bh_2[0:1, :], zero1)

        fs = fall2(bis1, bhs1, wis2, bis2, bhs2)
        fo = fall2(bio1, bho1, wio2, bio2, bho2)
        hS = jnp.where(any_s[...] > 0.0, hs2[...], fs)
        hO = jnp.where(any_o[...] > 0.0, ho2[...], fo)
        hcat = jnp.concatenate([hS, hO, h2[...]], axis=1)
        out_ref[...] = jnp.dot(hcat, fcw[...], preferred_element_type=f32) + fcb[...]


def kernel(context_features, params_inter, params_spk, params_oth, fc_w, fc_b,
           context_lengths, context_speaker_ids, roles):
    f32 = jnp.float32
    Bb, T, D = context_features.shape
    Hh = params_inter[0][1].shape[1]
    C = fc_w.shape[0]
    nc = T // CT

    x = jnp.transpose(context_features, (1, 0, 2)).astype(f32)

    lengths = jnp.asarray(context_lengths)
    sid = jnp.asarray(context_speaker_ids)
    roles_a = jnp.asarray(roles)
    t_idx = jnp.arange(T)
    valid = t_idx[:, None] < lengths[None, :]
    match = sid.T == roles_a[None, :]
    code = jnp.where(valid, jnp.where(match, 1.0, -1.0), 0.0).astype(f32)
    code_b = jnp.broadcast_to(code[:, :, None], (T, Bb, Hh))

    def prep(pr):
        W_ih, W_hh, b_ih, b_hh = pr
        return (W_ih.T.astype(jnp.bfloat16), W_hh.T.astype(jnp.bfloat16),
                jnp.broadcast_to(b_ih[None, :].astype(f32), (Bb, 3 * Hh)),
                jnp.broadcast_to(b_hh[None, :].astype(f32), (Bb, 3 * Hh)))

    layers = [prep(pr) for pr in (params_inter + params_spk + params_oth)]
    w_args = [a for lay in layers for a in lay]

    fcw_pad = jnp.zeros((3 * Hh, 128), f32).at[:, :C].set(fc_w.T.astype(f32))
    fcb_pad = jnp.broadcast_to(
        jnp.zeros((128,), f32).at[:C].set(fc_b.astype(f32))[None, :], (Bb, 128))

    def code_spec(k):
        return pl.BlockSpec(
            (CT, Bb, Hh), lambda c, k=k: (jnp.clip(c - k, 0, nc - 1), 0, 0))

    full2d = lambda a: pl.BlockSpec(a.shape, lambda c: (0, 0))
    in_specs = [
        pl.BlockSpec((CT, Bb, D), lambda c: (jnp.minimum(c, nc - 1), 0, 0)),
        code_spec(0), code_spec(1), code_spec(2), code_spec(3),
    ] + [full2d(a) for a in w_args] + [full2d(fcw_pad), full2d(fcb_pad)]

    scratch = (
        [pltpu.VMEM((CT, Bb, 3 * Hh), f32)] * 6
        + [pltpu.VMEM((2, CT, Bb, Hh), f32)] * 4
        + [pltpu.VMEM((Bb, Hh), f32)] * 8
    )

    body = functools.partial(_fused_body, Bb, Hh, nc)

    out = pl.pallas_call(
        body,
        grid=(nc + 3,),
        in_specs=in_specs,
        out_specs=pl.BlockSpec((Bb, 128), lambda c: (0, 0)),
        out_shape=jax.ShapeDtypeStruct((Bb, 128), f32),
        scratch_shapes=scratch,
        compiler_params=pltpu.CompilerParams(
            dimension_semantics=("arbitrary",),
            vmem_limit_bytes=100 * 1024 * 1024,
        ),
    )(x, code_b, code_b, code_b, code_b, *w_args, fcw_pad, fcb_pad)

    return out[:, :C]
